# baseline (speedup 1.0000x reference)
import functools

import jax
import jax.numpy as jnp
from jax import lax
from jax.experimental import pallas as pl
from jax.experimental.pallas import tpu as pltpu
from jax.experimental.pallas import tpu_sc as plsc

NC, NS, L = 2, 16, 16
NW = NC * NS
F32 = jnp.float32


def _sc_mesh():
    return plsc.VectorSubcoreMesh(
        core_axis_name="c", subcore_axis_name="s", num_cores=NC, num_subcores=NS
    )


_SC_PARAMS = pltpu.CompilerParams(needs_layout_passes=False)


def _sc_deg(dst, n_pad):
    E = dst.shape[0]
    nw = NC * NS
    ept = E // nw
    assert ept * nw == E
    nfull = ept // L
    rem = ept - nfull * L
    nit = nfull + (1 if rem else 0)
    buf_len = nit * L

    @functools.partial(
        pl.kernel,
        mesh=_sc_mesh(),
        out_type=jax.ShapeDtypeStruct((nw, n_pad), F32),
        compiler_params=_SC_PARAMS,
        scratch_types=[
            pltpu.VMEM((buf_len,), jnp.int32),
            pltpu.VMEM((n_pad,), F32),
        ],
    )
    def body(dst_hbm, part_hbm, dstb, degloc):
        cid = lax.axis_index("c")
        sid = lax.axis_index("s")
        wid = cid * NS + sid
        base = wid * ept

        def zrow(r, carry):
            degloc[pl.ds(r * L, L)] = jnp.zeros((L,), F32)
            return carry

        lax.fori_loop(0, n_pad // L, zrow, 0)
        if rem:
            dstb[pl.ds(nfull * L, L)] = jnp.zeros((L,), jnp.int32)
        pltpu.sync_copy(dst_hbm.at[pl.ds(base, ept)], dstb.at[pl.ds(0, ept)])

        ones = jnp.ones((L,), F32)

        def hist(j, carry):
            v = dstb[pl.ds(j * L, L)]
            plsc.addupdate_scatter(degloc, [v], ones)
            return carry

        lax.fori_loop(0, nfull, hist, 0)
        if rem:
            v = dstb[pl.ds(nfull * L, L)]
            m = lax.iota(jnp.int32, L) < rem
            plsc.addupdate_scatter(degloc, [v], ones, mask=m)

        pltpu.sync_copy(degloc, part_hbm.at[wid])

    return body(dst)


def _sc_agg(xs, src, dst, nch, n):
    E = src.shape[0]
    C = xs.shape[1]
    ch_per_core = nch // NC
    assert ch_per_core * NC == nch
    tps = E // NS
    assert tps * NS == E
    K = 80
    assert K % L == 0
    nblk = tps // K
    assert nblk * K == tps
    NI = 8
    NR = 4
    GROUP = 8
    MAXOFS = 4
    ngrp = (nblk + MAXOFS + GROUP - 1) // GROUP
    rpt = (n // NS) // 8 * 8
    rem_rows = n - rpt * NS
    assert rem_rows % 8 == 0

    @functools.partial(
        pl.kernel,
        mesh=_sc_mesh(),
        out_type=jax.ShapeDtypeStruct((nch * n, C), F32),
        compiler_params=_SC_PARAMS,
        scratch_types=(
            [pltpu.VMEM_SHARED((n, C), F32)]
            + [pltpu.VMEM((K,), jnp.int32) for _ in range(2 * NI)]
            + [pltpu.VMEM((K, C), F32) for _ in range(NR)]
            + [pltpu.SemaphoreType.DMA for _ in range(2 * NI + 2 * NR)]
        ),
    )
    def body(xs_hbm, src_hbm, dst_hbm, out_hbm, acc_sp, *bufs):
        idxb = bufs[0:NI]
        dstb = bufs[NI:2 * NI]
        rowsb = bufs[2 * NI:2 * NI + NR]
        o = 2 * NI + NR
        sem_i = bufs[o:o + 2 * NI]
        sem_g = bufs[o + 2 * NI:o + 2 * NI + NR]
        sem_s = bufs[o + 2 * NI + NR:o + 2 * NI + 2 * NR]
        cid = lax.axis_index("c")
        sid = lax.axis_index("s")
        ebase = sid * tps

        def issue(b, pi, pr):
            sl = pl.ds(ebase + b * K, K)
            pltpu.make_async_copy(src_hbm.at[sl], idxb[pi], sem_i[pi]).start()
            pltpu.make_async_copy(dst_hbm.at[sl], dstb[pi], sem_i[NI + pi]).start()

        def gath(b, pi, pr, off):
            sl = pl.ds(ebase + b * K, K)
            pltpu.make_async_copy(src_hbm.at[sl], idxb[pi], sem_i[pi]).wait()
            for j in range(K // L):
                vsl = pl.ds(j * L, L)
                idxb[pi][vsl] = idxb[pi][vsl] + off
            pltpu.make_async_copy(xs_hbm.at[idxb[pi]], rowsb[pr], sem_g[pr]).start()

        def scat(b, pi, pr):
            sl = pl.ds(ebase + b * K, K)
            pltpu.make_async_copy(xs_hbm.at[idxb[pi]], rowsb[pr], sem_g[pr]).wait()
            pltpu.make_async_copy(dst_hbm.at[sl], dstb[pi], sem_i[NI + pi]).wait()
            pltpu.async_copy(rowsb[pr], acc_sp.at[dstb[pi]], sem_s[pr], add=True)

        def fin(b, pi, pr):
            pltpu.make_async_copy(rowsb[pr], acc_sp.at[dstb[pi]], sem_s[pr]).wait()

        stages = ((fin, 4), (issue, 0), (gath, 1), (scat, 3))


        for k in range(ch_per_core):
            off = (cid * ch_per_core + k) * n

            pltpu.sync_copy(
                xs_hbm.at[pl.ds(off + sid * rpt, rpt)],
                acc_sp.at[pl.ds(sid * rpt, rpt)],
            )
            if rem_rows:
                @pl.when(sid == 0)
                def _():
                    pltpu.sync_copy(
                        xs_hbm.at[pl.ds(off + NS * rpt, rem_rows)],
                        acc_sp.at[pl.ds(NS * rpt, rem_rows)],
                    )

            plsc.subcore_barrier()

            def grp(g, carry):
                for p in range(GROUP):
                    for fn, ofs in stages:
                        b = g * GROUP + p - ofs
                        pi = (p - ofs) % NI
                        pr = (p - ofs) % NR

                        @pl.when(jnp.logical_and(b >= 0, b < nblk))
                        def _(fn=fn, b=b, pi=pi, pr=pr):
                            if fn is gath:
                                fn(b, pi, pr, off)
                            else:
                                fn(b, pi, pr)

                return carry

            lax.fori_loop(0, ngrp, grp, 0)
            plsc.subcore_barrier()

            pltpu.sync_copy(
                acc_sp.at[pl.ds(sid * rpt, rpt)],
                out_hbm.at[pl.ds(off + sid * rpt, rpt)],
            )
            if rem_rows:
                @pl.when(sid == 0)
                def _():
                    pltpu.sync_copy(
                        acc_sp.at[pl.ds(NS * rpt, rem_rows)],
                        out_hbm.at[pl.ds(off + NS * rpt, rem_rows)],
                    )

    return body(xs, src, dst)


def _tc_m1(x, w1, parr, bm):
    n, din = x.shape
    dout = w1.shape[1]
    nch = dout // 128
    nbi = n // bm

    def m1_body(x_ref, w_ref, p_ref, o_ref):
        p = p_ref[0]
        dis = lax.rsqrt(jnp.sum(p, axis=0) + 1.0)
        xw = jnp.dot(x_ref[...], w_ref[...], preferred_element_type=F32)
        xs = xw * dis[:, None]
        for c in range(nch):
            o_ref[c] = xs[:, c * 128:(c + 1) * 128]

    return pl.pallas_call(
        m1_body,
        grid=(nbi,),
        in_specs=[
            pl.BlockSpec((bm, din), lambda i: (i, 0)),
            pl.BlockSpec((din, dout), lambda i: (0, 0)),
            pl.BlockSpec((1, NW, bm), lambda i: (i, 0, 0)),
        ],
        out_specs=pl.BlockSpec((nch, bm, 128), lambda i: (0, i, 0)),
        out_shape=jax.ShapeDtypeStruct((nch, n, 128), F32),
    )(x, w1, parr)


def _tc_m2(acc1, parr, b1r, w2, n, bm):
    nch_in = b1r.shape[0]
    d_out = w2.shape[1]
    nch_out = d_out // 128
    nbi = n // bm

    def m2_body(a_ref, p_ref, b_ref, w_ref, o_ref, yacc):
        c = pl.program_id(1)
        p = p_ref[0]
        dis = lax.rsqrt(jnp.sum(p, axis=0) + 1.0)
        h = jnp.maximum(a_ref[0] * dis[:, None] + b_ref[0, 0], 0.0)
        y = jnp.dot(h, w_ref[...], preferred_element_type=F32)

        @pl.when(c == 0)
        def _():
            yacc[...] = jnp.zeros_like(yacc)

        yacc[...] += y

        @pl.when(c == nch_in - 1)
        def _():
            ys = yacc[...] * dis[:, None]
            for c2 in range(nch_out):
                o_ref[c2] = ys[:, c2 * 128:(c2 + 1) * 128]

    return pl.pallas_call(
        m2_body,
        grid=(nbi, nch_in),
        in_specs=[
            pl.BlockSpec((1, bm, 128), lambda i, c: (c, i, 0)),
            pl.BlockSpec((1, NW, bm), lambda i, c: (i, 0, 0)),
            pl.BlockSpec((1, 1, 128), lambda i, c: (c, 0, 0)),
            pl.BlockSpec((128, d_out), lambda i, c: (c, 0)),
        ],
        out_specs=pl.BlockSpec((nch_out, bm, 128), lambda i, c: (0, i, 0)),
        out_shape=jax.ShapeDtypeStruct((nch_out, n, 128), F32),
        scratch_shapes=[pltpu.VMEM((bm, d_out), F32)],
    )(acc1, parr, b1r, w2)


def _tc_f(acc2, parr, b2r, n, bm):
    nch = b2r.shape[0]
    nbi = n // bm

    def f_body(a_ref, p_ref, b_ref, o_ref):
        p = p_ref[0]
        dis = lax.rsqrt(jnp.sum(p, axis=0) + 1.0)
        for c2 in range(nch):
            o_ref[:, c2 * 128:(c2 + 1) * 128] = (
                a_ref[c2] * dis[:, None] + b_ref[c2, 0]
            )

    return pl.pallas_call(
        f_body,
        grid=(nbi,),
        in_specs=[
            pl.BlockSpec((nch, bm, 128), lambda i: (0, i, 0)),
            pl.BlockSpec((1, NW, bm), lambda i: (i, 0, 0)),
            pl.BlockSpec((nch, 1, 128), lambda i: (0, 0, 0)),
        ],
        out_specs=pl.BlockSpec((bm, nch * 128), lambda i: (i, 0)),
        out_shape=jax.ShapeDtypeStruct((n, nch * 128), F32),
    )(acc2, parr, b2r)


def kernel(x, edge_index, W1, b1, W2, b2):
    n, din = x.shape
    d_hid = W1.shape[1]
    d_out = W2.shape[1]
    nch1 = d_hid // 128
    nch2 = d_out // 128
    assert nch1 * 128 == d_hid and nch2 * 128 == d_out

    src = edge_index[0].astype(jnp.int32)
    dst = edge_index[1].astype(jnp.int32)

    n_pad = ((n + 2047) // 2048) * 2048
    partial = _sc_deg(dst, n_pad)

    bm = 5000
    nbi = n // bm
    assert nbi * bm == n
    p2 = partial[:, :n]
    parr = p2.reshape(NW, nbi, bm).transpose(1, 0, 2)

    xs1 = _tc_m1(x, W1, parr, bm)
    acc1 = _sc_agg(xs1.reshape(nch1 * n, 128), src, dst, nch1, n)

    xs2 = _tc_m2(
        acc1.reshape(nch1, n, 128), parr, b1.reshape(nch1, 1, 128), W2, n, bm
    )
    acc2 = _sc_agg(xs2.reshape(nch2 * n, 128), src, dst, nch2, n)

    return _tc_f(acc2.reshape(nch2, n, 128), parr, b2.reshape(nch2, 1, 128), n, bm)


# speedup vs baseline: 1.0044x; 1.0044x over previous
import functools

import jax
import jax.numpy as jnp
from jax import lax
from jax.experimental import pallas as pl
from jax.experimental.pallas import tpu as pltpu
from jax.experimental.pallas import tpu_sc as plsc

NC, NS, L = 2, 16, 16
NW = NC * NS
F32 = jnp.float32


def _sc_mesh():
    return plsc.VectorSubcoreMesh(
        core_axis_name="c", subcore_axis_name="s", num_cores=NC, num_subcores=NS
    )


_SC_PARAMS = pltpu.CompilerParams(needs_layout_passes=False)


def _sc_deg(dst, n_pad):
    E = dst.shape[0]
    nw = NC * NS
    ept = E // nw
    assert ept * nw == E
    nfull = ept // L
    rem = ept - nfull * L
    nit = nfull + (1 if rem else 0)
    buf_len = nit * L

    @functools.partial(
        pl.kernel,
        mesh=_sc_mesh(),
        out_type=jax.ShapeDtypeStruct((nw, n_pad), F32),
        compiler_params=_SC_PARAMS,
        scratch_types=[
            pltpu.VMEM((buf_len,), jnp.int32),
            pltpu.VMEM((n_pad,), F32),
        ],
    )
    def body(dst_hbm, part_hbm, dstb, degloc):
        cid = lax.axis_index("c")
        sid = lax.axis_index("s")
        wid = cid * NS + sid
        base = wid * ept

        def zrow(r, carry):
            degloc[pl.ds(r * L, L)] = jnp.zeros((L,), F32)
            return carry

        lax.fori_loop(0, n_pad // L, zrow, 0)
        if rem:
            dstb[pl.ds(nfull * L, L)] = jnp.zeros((L,), jnp.int32)
        pltpu.sync_copy(dst_hbm.at[pl.ds(base, ept)], dstb.at[pl.ds(0, ept)])

        ones = jnp.ones((L,), F32)

        def hist(j, carry):
            v = dstb[pl.ds(j * L, L)]
            plsc.addupdate_scatter(degloc, [v], ones)
            return carry

        lax.fori_loop(0, nfull, hist, 0)
        if rem:
            v = dstb[pl.ds(nfull * L, L)]
            m = lax.iota(jnp.int32, L) < rem
            plsc.addupdate_scatter(degloc, [v], ones, mask=m)

        pltpu.sync_copy(degloc, part_hbm.at[wid])

    return body(dst)


def _sc_agg(xs, src, dst, nch, n):
    E = src.shape[0]
    C = xs.shape[1]
    ch_per_core = nch // NC
    assert ch_per_core * NC == nch
    tps = E // NS
    assert tps * NS == E
    K = 80
    assert K % L == 0
    nblk = tps // K
    assert nblk * K == tps
    NI = 8
    NR = 4
    GROUP = 8
    MAXOFS = 4
    ngrp = (nblk + MAXOFS + GROUP - 1) // GROUP
    rpt = next(c for c in range(8, n + 1, 8) if n % c == 0 and n // c <= NS)
    nti = n // rpt

    @functools.partial(
        pl.kernel,
        mesh=_sc_mesh(),
        out_type=jax.ShapeDtypeStruct((nch * n, C), F32),
        compiler_params=_SC_PARAMS,
        scratch_types=(
            [pltpu.VMEM_SHARED((n, C), F32)]
            + [pltpu.VMEM((K,), jnp.int32) for _ in range(2 * NI)]
            + [pltpu.VMEM((K, C), F32) for _ in range(NR)]
            + [pltpu.SemaphoreType.DMA for _ in range(2 * NI + 2 * NR)]
        ),
    )
    def body(xs_hbm, src_hbm, dst_hbm, out_hbm, acc_sp, *bufs):
        idxb = bufs[0:NI]
        dstb = bufs[NI:2 * NI]
        rowsb = bufs[2 * NI:2 * NI + NR]
        o = 2 * NI + NR
        sem_i = bufs[o:o + 2 * NI]
        sem_g = bufs[o + 2 * NI:o + 2 * NI + NR]
        sem_s = bufs[o + 2 * NI + NR:o + 2 * NI + 2 * NR]
        cid = lax.axis_index("c")
        sid = lax.axis_index("s")
        ebase = sid * tps

        def issue(b, pi, pr):
            sl = pl.ds(ebase + b * K, K)
            pltpu.make_async_copy(src_hbm.at[sl], idxb[pi], sem_i[pi]).start()
            pltpu.make_async_copy(dst_hbm.at[sl], dstb[pi], sem_i[NI + pi]).start()

        def gath(b, pi, pr, off):
            sl = pl.ds(ebase + b * K, K)
            pltpu.make_async_copy(src_hbm.at[sl], idxb[pi], sem_i[pi]).wait()
            for j in range(K // L):
                vsl = pl.ds(j * L, L)
                idxb[pi][vsl] = idxb[pi][vsl] + off
            pltpu.make_async_copy(xs_hbm.at[idxb[pi]], rowsb[pr], sem_g[pr]).start()

        def scat(b, pi, pr):
            sl = pl.ds(ebase + b * K, K)
            pltpu.make_async_copy(xs_hbm.at[idxb[pi]], rowsb[pr], sem_g[pr]).wait()
            pltpu.make_async_copy(dst_hbm.at[sl], dstb[pi], sem_i[NI + pi]).wait()
            pltpu.async_copy(rowsb[pr], acc_sp.at[dstb[pi]], sem_s[pr], add=True)

        def fin(b, pi, pr):
            pltpu.make_async_copy(rowsb[pr], acc_sp.at[dstb[pi]], sem_s[pr]).wait()

        stages = ((fin, 4), (issue, 0), (gath, 1), (scat, 3))


        for k in range(ch_per_core):
            off = (cid * ch_per_core + k) * n

            @pl.when(sid < nti)
            def _():
                pltpu.sync_copy(
                    xs_hbm.at[pl.ds(off + sid * rpt, rpt)],
                    acc_sp.at[pl.ds(sid * rpt, rpt)],
                )

            plsc.subcore_barrier()

            def grp(g, carry):
                for p in range(GROUP):
                    for fn, ofs in stages:
                        b = g * GROUP + p - ofs
                        pi = (p - ofs) % NI
                        pr = (p - ofs) % NR

                        @pl.when(jnp.logical_and(b >= 0, b < nblk))
                        def _(fn=fn, b=b, pi=pi, pr=pr):
                            if fn is gath:
                                fn(b, pi, pr, off)
                            else:
                                fn(b, pi, pr)

                return carry

            lax.fori_loop(0, ngrp, grp, 0)
            plsc.subcore_barrier()

            @pl.when(sid < nti)
            def _():
                pltpu.sync_copy(
                    acc_sp.at[pl.ds(sid * rpt, rpt)],
                    out_hbm.at[pl.ds(off + sid * rpt, rpt)],
                )

    return body(xs, src, dst)


def _tc_m1(x, w1, parr, bm):
    n, din = x.shape
    dout = w1.shape[1]
    nch = dout // 128
    nbi = n // bm

    def m1_body(x_ref, w_ref, p_ref, o_ref):
        p = p_ref[0]
        dis = lax.rsqrt(jnp.sum(p, axis=0) + 1.0)
        xw = jnp.dot(x_ref[...], w_ref[...], preferred_element_type=F32)
        xs = xw * dis[:, None]
        for c in range(nch):
            o_ref[c] = xs[:, c * 128:(c + 1) * 128]

    return pl.pallas_call(
        m1_body,
        grid=(nbi,),
        in_specs=[
            pl.BlockSpec((bm, din), lambda i: (i, 0)),
            pl.BlockSpec((din, dout), lambda i: (0, 0)),
            pl.BlockSpec((1, NW, bm), lambda i: (i, 0, 0)),
        ],
        out_specs=pl.BlockSpec((nch, bm, 128), lambda i: (0, i, 0)),
        out_shape=jax.ShapeDtypeStruct((nch, n, 128), F32),
    )(x, w1, parr)


def _tc_m2(acc1, parr, b1r, w2, n, bm):
    nch_in = b1r.shape[0]
    d_out = w2.shape[1]
    nch_out = d_out // 128
    nbi = n // bm

    def m2_body(a_ref, p_ref, b_ref, w_ref, o_ref, yacc):
        c = pl.program_id(1)
        p = p_ref[0]
        dis = lax.rsqrt(jnp.sum(p, axis=0) + 1.0)
        h = jnp.maximum(a_ref[0] * dis[:, None] + b_ref[0, 0], 0.0)
        y = jnp.dot(h, w_ref[...], preferred_element_type=F32)

        @pl.when(c == 0)
        def _():
            yacc[...] = jnp.zeros_like(yacc)

        yacc[...] += y

        @pl.when(c == nch_in - 1)
        def _():
            ys = yacc[...] * dis[:, None]
            for c2 in range(nch_out):
                o_ref[c2] = ys[:, c2 * 128:(c2 + 1) * 128]

    return pl.pallas_call(
        m2_body,
        grid=(nbi, nch_in),
        in_specs=[
            pl.BlockSpec((1, bm, 128), lambda i, c: (c, i, 0)),
            pl.BlockSpec((1, NW, bm), lambda i, c: (i, 0, 0)),
            pl.BlockSpec((1, 1, 128), lambda i, c: (c, 0, 0)),
            pl.BlockSpec((128, d_out), lambda i, c: (c, 0)),
        ],
        out_specs=pl.BlockSpec((nch_out, bm, 128), lambda i, c: (0, i, 0)),
        out_shape=jax.ShapeDtypeStruct((nch_out, n, 128), F32),
        scratch_shapes=[pltpu.VMEM((bm, d_out), F32)],
    )(acc1, parr, b1r, w2)


def _tc_f(acc2, parr, b2r, n, bm):
    nch = b2r.shape[0]
    nbi = n // bm

    def f_body(a_ref, p_ref, b_ref, o_ref):
        p = p_ref[0]
        dis = lax.rsqrt(jnp.sum(p, axis=0) + 1.0)
        for c2 in range(nch):
            o_ref[:, c2 * 128:(c2 + 1) * 128] = (
                a_ref[c2] * dis[:, None] + b_ref[c2, 0]
            )

    return pl.pallas_call(
        f_body,
        grid=(nbi,),
        in_specs=[
            pl.BlockSpec((nch, bm, 128), lambda i: (0, i, 0)),
            pl.BlockSpec((1, NW, bm), lambda i: (i, 0, 0)),
            pl.BlockSpec((nch, 1, 128), lambda i: (0, 0, 0)),
        ],
        out_specs=pl.BlockSpec((bm, nch * 128), lambda i: (i, 0)),
        out_shape=jax.ShapeDtypeStruct((n, nch * 128), F32),
    )(acc2, parr, b2r)


def kernel(x, edge_index, W1, b1, W2, b2):
    n, din = x.shape
    d_hid = W1.shape[1]
    d_out = W2.shape[1]
    nch1 = d_hid // 128
    nch2 = d_out // 128
    assert nch1 * 128 == d_hid and nch2 * 128 == d_out

    src = edge_index[0].astype(jnp.int32)
    dst = edge_index[1].astype(jnp.int32)

    n_pad = ((n + 2047) // 2048) * 2048
    partial = _sc_deg(dst, n_pad)

    bm = 5000
    nbi = n // bm
    assert nbi * bm == n
    p2 = partial[:, :n]
    parr = p2.reshape(NW, nbi, bm).transpose(1, 0, 2)

    xs1 = _tc_m1(x, W1, parr, bm)
    acc1 = _sc_agg(xs1.reshape(nch1 * n, 128), src, dst, nch1, n)

    xs2 = _tc_m2(
        acc1.reshape(nch1, n, 128), parr, b1.reshape(nch1, 1, 128), W2, n, bm
    )
    acc2 = _sc_agg(xs2.reshape(nch2 * n, 128), src, dst, nch2, n)

    return _tc_f(acc2.reshape(nch2, n, 128), parr, b2.reshape(nch2, 1, 128), n, bm)
